# baseline (speedup 1.0000x reference)
import functools

import jax
import jax.numpy as jnp
from jax import lax
from jax.experimental import pallas as pl
from jax.experimental.pallas import tpu as pltpu
from jax.experimental.pallas import tpu_sc as plsc

NC, NS, LANES = 2, 16, 16
NW = NC * NS

B, L = 16384, 50
D = 32
RPW = B // NW
CHUNKS = RPW // LANES
KROWS = (RPW * L) // 128


def _dot_body(e_ref, w_ref, o_ref):
    o_ref[:, :64] = jnp.sum(e_ref[...] * w_ref[...][None], axis=2)


SGRP = 64
VOCAB = 1000000
SROWS_V = VOCAB // SGRP
RB = 8
SGRID = (SROWS_V + RB - 1) // RB
SROWS = SGRID * RB


def _precompute_s(embed, W):
    e3 = embed.reshape(SROWS_V, SGRP, D)
    s2 = pl.pallas_call(
        _dot_body,
        grid=(SGRID,),
        in_specs=[
            pl.BlockSpec((RB, SGRP, D), lambda i: (i, 0, 0)),
            pl.BlockSpec((1, D), lambda i: (0, 0)),
        ],
        out_specs=pl.BlockSpec((RB, 128), lambda i: (i, 0)),
        out_shape=jax.ShapeDtypeStruct((SROWS, 128), jnp.float32),
    )(e3, W)
    return s2.reshape(SROWS * 128)


def _sc_body(s_hbm, idx_hbm, mask_hbm, b_hbm, out_hbm,
             idx_v, mask_v, vals_v, out_v, b_v, sem):
    wid = lax.axis_index("s") * NC + lax.axis_index("c")
    pltpu.sync_copy(idx_hbm.at[wid], idx_v)
    pltpu.sync_copy(mask_hbm.at[wid], mask_v)
    pltpu.sync_copy(b_hbm, b_v)

    def remap(i, _):
        for u in range(8):
            off = (i * 8 + u) * LANES
            v = idx_v[pl.ds(off, LANES)]
            idx_v[pl.ds(off, LANES)] = (
                ((v >> 6) << 7) | (v & 63))
        return 0
    lax.fori_loop(0, (RPW * L) // (8 * LANES), remap, 0)

    pltpu.async_copy(s_hbm.at[idx_v], vals_v, sem).wait()

    bias = b_v[...]
    zero = jnp.zeros((LANES,), jnp.float32)
    stride = lax.iota(jnp.int32, LANES) * L
    for c in range(CHUNKS):
        base = c * LANES * L
        def body(l, carry, base=base):
            acc, msum = carry
            gidx = stride + (base + l)
            v = plsc.load_gather(vals_v, [gidx])
            m = plsc.load_gather(mask_v, [gidx])
            return acc + v * m, msum + m
        acc, msum = lax.fori_loop(0, L, body, (zero, zero))
        out_v[pl.ds(c * LANES, LANES)] = acc / (msum + 1e-9) + bias
    pltpu.sync_copy(out_v, out_hbm.at[pl.ds(wid * RPW, RPW)])


@functools.cache
def _make_sc_call():
    mesh = plsc.VectorSubcoreMesh(
        core_axis_name="c", subcore_axis_name="s",
        num_cores=NC, num_subcores=NS)
    return pl.kernel(
        _sc_body,
        out_type=jax.ShapeDtypeStruct((B,), jnp.float32),
        mesh=mesh,
        compiler_params=pltpu.CompilerParams(needs_layout_passes=False),
        scratch_types=[
            pltpu.VMEM((RPW * L,), jnp.int32),
            pltpu.VMEM((RPW * L,), jnp.float32),
            pltpu.VMEM((RPW * L,), jnp.float32),
            pltpu.VMEM((RPW,), jnp.float32),
            pltpu.VMEM((LANES,), jnp.float32),
            pltpu.SemaphoreType.DMA,
        ],
    )


@jax.jit
def kernel(inputs, mask, embed, W, b):
    s = _precompute_s(embed.astype(jnp.float32), W.astype(jnp.float32))
    idx_t = inputs.astype(jnp.int32).reshape(NW, RPW * L)
    mask_t = mask.astype(jnp.float32).reshape(NW, RPW * L)
    b16 = jnp.broadcast_to(b.astype(jnp.float32).reshape(()), (LANES,))
    return _make_sc_call()(s, idx_t, mask_t, b16)


# speedup vs baseline: 3.0623x; 3.0623x over previous
import functools

import jax
import jax.numpy as jnp
from jax import lax
from jax.experimental import pallas as pl
from jax.experimental.pallas import tpu as pltpu
from jax.experimental.pallas import tpu_sc as plsc

NC, NS, LANES = 2, 16, 16
NW = NC * NS

B, L = 16384, 50
D = 32
RPW = B // NW
CHUNKS = RPW // LANES
KROWS = (RPW * L) // 128


def _dot_body(e_ref, w_ref, o_ref):
    o_ref[...] = jnp.sum(e_ref[...] * w_ref[...][None], axis=2)


def _precompute_s(embed, W):
    V = embed.shape[0]
    C = 1000
    R = V // C
    RB = 8
    e3 = embed.reshape(R, C, D)
    s2 = pl.pallas_call(
        _dot_body,
        grid=(R // RB,),
        in_specs=[
            pl.BlockSpec((RB, C, D), lambda i: (i, 0, 0)),
            pl.BlockSpec((1, D), lambda i: (0, 0)),
        ],
        out_specs=pl.BlockSpec((RB, C), lambda i: (i, 0)),
        out_shape=jax.ShapeDtypeStruct((R, C), jnp.float32),
    )(e3, W)
    return s2.reshape(V)


SBLK = 1024
LP = 64


def _stage_body(i_ref, m_ref, oi_ref, om_ref):
    zi = jnp.zeros((SBLK, LP - L), jnp.int32)
    zm = jnp.zeros((SBLK, LP - L), jnp.float32)
    oi_ref[...] = jnp.concatenate([i_ref[...], zi], axis=1).T
    om_ref[...] = jnp.concatenate([m_ref[...], zm], axis=1).T


def _stage_t(inputs, mask):
    grid = B // SBLK
    idxf, maskf = pl.pallas_call(
        _stage_body,
        grid=(grid,),
        in_specs=[
            pl.BlockSpec((SBLK, L), lambda i: (i, 0)),
            pl.BlockSpec((SBLK, L), lambda i: (i, 0)),
        ],
        out_specs=[
            pl.BlockSpec((LP, SBLK), lambda i: (0, i)),
            pl.BlockSpec((LP, SBLK), lambda i: (0, i)),
        ],
        out_shape=[
            jax.ShapeDtypeStruct((LP, B), jnp.int32),
            jax.ShapeDtypeStruct((LP, B), jnp.float32),
        ],
    )(inputs, mask)
    return idxf, maskf


def _sc_body(s_hbm, idx_hbm, mask_hbm, b_hbm, out_hbm,
             idx_v, mask_v, vals_v, out_v, b_v, sem):
    wid = lax.axis_index("s") * NC + lax.axis_index("c")
    base = wid * RPW
    copies = []
    for l in range(L):
        copies.append(pltpu.async_copy(
            idx_hbm.at[l, pl.ds(base, RPW)],
            idx_v.at[pl.ds(l * RPW, RPW)], sem))
        copies.append(pltpu.async_copy(
            mask_hbm.at[l, pl.ds(base, RPW)],
            mask_v.at[pl.ds(l * RPW, RPW)], sem))
    pltpu.sync_copy(b_hbm, b_v)
    for cp in copies:
        cp.wait()
    pltpu.async_copy(s_hbm.at[idx_v], vals_v, sem).wait()

    bias = b_v[...]
    zero = jnp.zeros((LANES,), jnp.float32)
    for c in range(CHUNKS):
        col = c * LANES
        def body(l, carry, col=col):
            acc, msum = carry
            off = l * RPW + col
            v = vals_v[pl.ds(off, LANES)]
            m = mask_v[pl.ds(off, LANES)]
            return acc + v * m, msum + m
        acc, msum = lax.fori_loop(0, L, body, (zero, zero))
        out_v[pl.ds(c * LANES, LANES)] = acc / (msum + 1e-9) + bias
    pltpu.sync_copy(out_v, out_hbm.at[pl.ds(wid * RPW, RPW)])


@functools.cache
def _make_sc_call():
    mesh = plsc.VectorSubcoreMesh(
        core_axis_name="c", subcore_axis_name="s",
        num_cores=NC, num_subcores=NS)
    return pl.kernel(
        _sc_body,
        out_type=jax.ShapeDtypeStruct((B,), jnp.float32),
        mesh=mesh,
        compiler_params=pltpu.CompilerParams(needs_layout_passes=False),
        scratch_types=[
            pltpu.VMEM((L * RPW,), jnp.int32),
            pltpu.VMEM((L * RPW,), jnp.float32),
            pltpu.VMEM((L * RPW,), jnp.float32),
            pltpu.VMEM((RPW,), jnp.float32),
            pltpu.VMEM((LANES,), jnp.float32),
            pltpu.SemaphoreType.DMA,
        ],
    )


@jax.jit
def kernel(inputs, mask, embed, W, b):
    s = _precompute_s(embed.astype(jnp.float32), W.astype(jnp.float32))
    idx_t, mask_t = _stage_t(
        inputs.astype(jnp.int32), mask.astype(jnp.float32))
    b16 = jnp.broadcast_to(b.astype(jnp.float32).reshape(()), (LANES,))
    return _make_sc_call()(s, idx_t, mask_t, b16)


# speedup vs baseline: 3.0730x; 1.0035x over previous
import functools

import jax
import jax.numpy as jnp
from jax import lax
from jax.experimental import pallas as pl
from jax.experimental.pallas import tpu as pltpu
from jax.experimental.pallas import tpu_sc as plsc

NC, NS, LANES = 2, 16, 16
NW = NC * NS

B, L = 16384, 50
D = 32
RPW = B // NW
CHUNKS = RPW // LANES
KROWS = (RPW * L) // 128


def _dot_body(e_ref, w_ref, o_ref):
    o_ref[:, :1000] = jnp.sum(e_ref[...] * w_ref[...][None], axis=2)


def _precompute_s(embed, W):
    V = embed.shape[0]
    C = 1000
    R = V // C
    RB = 8
    e3 = embed.reshape(R, C, D)
    s2 = pl.pallas_call(
        _dot_body,
        grid=(R // RB,),
        in_specs=[
            pl.BlockSpec((RB, C, D), lambda i: (i, 0, 0)),
            pl.BlockSpec((1, D), lambda i: (0, 0)),
        ],
        out_specs=pl.BlockSpec((RB, 1024), lambda i: (i, 0)),
        out_shape=jax.ShapeDtypeStruct((R, 1024), jnp.float32),
    )(e3, W)
    return s2.reshape(R * 1024)


SBLK = 1024
LP = 64


def _stage_body(i_ref, m_ref, oi_ref, om_ref):
    zi = jnp.zeros((SBLK, LP - L), jnp.int32)
    zm = jnp.zeros((SBLK, LP - L), jnp.float32)
    it = jnp.concatenate([i_ref[...], zi], axis=1).T
    oi_ref[...] = it + (it // 1000) * 24
    om_ref[...] = jnp.concatenate([m_ref[...], zm], axis=1).T


def _stage_t(inputs, mask):
    grid = B // SBLK
    idxf, maskf = pl.pallas_call(
        _stage_body,
        grid=(grid,),
        in_specs=[
            pl.BlockSpec((SBLK, L), lambda i: (i, 0)),
            pl.BlockSpec((SBLK, L), lambda i: (i, 0)),
        ],
        out_specs=[
            pl.BlockSpec((LP, SBLK), lambda i: (0, i)),
            pl.BlockSpec((LP, SBLK), lambda i: (0, i)),
        ],
        out_shape=[
            jax.ShapeDtypeStruct((LP, B), jnp.int32),
            jax.ShapeDtypeStruct((LP, B), jnp.float32),
        ],
    )(inputs, mask)
    return idxf, maskf


def _sc_body(s_hbm, idx_hbm, mask_hbm, b_hbm, out_hbm,
             idx_v, mask_v, vals_v, out_v, b_v, sem):
    wid = lax.axis_index("s") * NC + lax.axis_index("c")
    base = wid * RPW
    copies = []
    for l in range(L):
        copies.append(pltpu.async_copy(
            idx_hbm.at[l, pl.ds(base, RPW)],
            idx_v.at[pl.ds(l * RPW, RPW)], sem))
        copies.append(pltpu.async_copy(
            mask_hbm.at[l, pl.ds(base, RPW)],
            mask_v.at[pl.ds(l * RPW, RPW)], sem))
    pltpu.sync_copy(b_hbm, b_v)
    for cp in copies:
        cp.wait()
    pltpu.async_copy(s_hbm.at[idx_v], vals_v, sem).wait()

    bias = b_v[...]
    zero = jnp.zeros((LANES,), jnp.float32)
    for c in range(CHUNKS):
        col = c * LANES
        def body(l, carry, col=col):
            acc, msum = carry
            off = l * RPW + col
            v = vals_v[pl.ds(off, LANES)]
            m = mask_v[pl.ds(off, LANES)]
            return acc + v * m, msum + m
        acc, msum = lax.fori_loop(0, L, body, (zero, zero))
        out_v[pl.ds(c * LANES, LANES)] = acc / (msum + 1e-9) + bias
    pltpu.sync_copy(out_v, out_hbm.at[pl.ds(wid * RPW, RPW)])


@functools.cache
def _make_sc_call():
    mesh = plsc.VectorSubcoreMesh(
        core_axis_name="c", subcore_axis_name="s",
        num_cores=NC, num_subcores=NS)
    return pl.kernel(
        _sc_body,
        out_type=jax.ShapeDtypeStruct((B,), jnp.float32),
        mesh=mesh,
        compiler_params=pltpu.CompilerParams(needs_layout_passes=False),
        scratch_types=[
            pltpu.VMEM((L * RPW,), jnp.int32),
            pltpu.VMEM((L * RPW,), jnp.float32),
            pltpu.VMEM((L * RPW,), jnp.float32),
            pltpu.VMEM((RPW,), jnp.float32),
            pltpu.VMEM((LANES,), jnp.float32),
            pltpu.SemaphoreType.DMA,
        ],
    )


@jax.jit
def kernel(inputs, mask, embed, W, b):
    s = _precompute_s(embed.astype(jnp.float32), W.astype(jnp.float32))
    idx_t, mask_t = _stage_t(
        inputs.astype(jnp.int32), mask.astype(jnp.float32))
    b16 = jnp.broadcast_to(b.astype(jnp.float32).reshape(()), (LANES,))
    return _make_sc_call()(s, idx_t, mask_t, b16)


# speedup vs baseline: 3.3778x; 1.0992x over previous
import functools

import jax
import jax.numpy as jnp
from jax import lax
from jax.experimental import pallas as pl
from jax.experimental.pallas import tpu as pltpu
from jax.experimental.pallas import tpu_sc as plsc

NC, NS, LANES = 2, 16, 16
NW = NC * NS

B, L = 16384, 50
D = 32
RPW = B // NW
CHUNKS = RPW // LANES
KROWS = (RPW * L) // 128


def _dot_body(e_ref, w_ref, o_ref):
    o_ref[:, :64] = jnp.sum(e_ref[...] * w_ref[...][None], axis=2)


SGRP = 64
SROWS_V = 1000000 // SGRP
SRB = 256
SGRID = (SROWS_V + SRB - 1) // SRB
SROWS = SGRID * SRB


def _precompute_s(embed, W):
    e3 = embed.reshape(SROWS_V, SGRP, D)
    s2 = pl.pallas_call(
        _dot_body,
        grid=(SGRID,),
        in_specs=[
            pl.BlockSpec((SRB, SGRP, D), lambda i: (i, 0, 0)),
            pl.BlockSpec((1, D), lambda i: (0, 0)),
        ],
        out_specs=pl.BlockSpec((SRB, 128), lambda i: (i, 0)),
        out_shape=jax.ShapeDtypeStruct((SROWS, 128), jnp.float32),
    )(e3, W)
    return s2.reshape(SROWS * 128)


SBLK = 1024
LP = 64


def _stage_body(i_ref, m_ref, oi_ref, om_ref):
    zi = jnp.zeros((SBLK, LP - L), jnp.int32)
    zm = jnp.zeros((SBLK, LP - L), jnp.float32)
    it = jnp.concatenate([i_ref[...], zi], axis=1).T
    oi_ref[...] = ((it >> 6) << 7) | (it & 63)
    om_ref[...] = jnp.concatenate([m_ref[...], zm], axis=1).T


def _stage_t(inputs, mask):
    grid = B // SBLK
    idxf, maskf = pl.pallas_call(
        _stage_body,
        grid=(grid,),
        in_specs=[
            pl.BlockSpec((SBLK, L), lambda i: (i, 0)),
            pl.BlockSpec((SBLK, L), lambda i: (i, 0)),
        ],
        out_specs=[
            pl.BlockSpec((LP, SBLK), lambda i: (0, i)),
            pl.BlockSpec((LP, SBLK), lambda i: (0, i)),
        ],
        out_shape=[
            jax.ShapeDtypeStruct((LP, B), jnp.int32),
            jax.ShapeDtypeStruct((LP, B), jnp.float32),
        ],
    )(inputs, mask)
    return idxf, maskf


def _sc_body(s_hbm, idx_hbm, mask_hbm, b_hbm, out_hbm,
             idx_v, mask_v, vals_v, out_v, b_v, sem):
    wid = lax.axis_index("s") * NC + lax.axis_index("c")
    base = wid * RPW
    copies = []
    for l in range(L):
        copies.append(pltpu.async_copy(
            idx_hbm.at[l, pl.ds(base, RPW)],
            idx_v.at[pl.ds(l * RPW, RPW)], sem))
        copies.append(pltpu.async_copy(
            mask_hbm.at[l, pl.ds(base, RPW)],
            mask_v.at[pl.ds(l * RPW, RPW)], sem))
    pltpu.sync_copy(b_hbm, b_v)
    for cp in copies:
        cp.wait()
    pltpu.async_copy(s_hbm.at[idx_v], vals_v, sem).wait()

    bias = b_v[...]
    zero = jnp.zeros((LANES,), jnp.float32)
    for c in range(CHUNKS):
        col = c * LANES
        def body(l, carry, col=col):
            acc, msum = carry
            off = l * RPW + col
            v = vals_v[pl.ds(off, LANES)]
            m = mask_v[pl.ds(off, LANES)]
            return acc + v * m, msum + m
        acc, msum = lax.fori_loop(0, L, body, (zero, zero))
        out_v[pl.ds(c * LANES, LANES)] = acc / (msum + 1e-9) + bias
    pltpu.sync_copy(out_v, out_hbm.at[pl.ds(wid * RPW, RPW)])


@functools.cache
def _make_sc_call():
    mesh = plsc.VectorSubcoreMesh(
        core_axis_name="c", subcore_axis_name="s",
        num_cores=NC, num_subcores=NS)
    return pl.kernel(
        _sc_body,
        out_type=jax.ShapeDtypeStruct((B,), jnp.float32),
        mesh=mesh,
        compiler_params=pltpu.CompilerParams(needs_layout_passes=False),
        scratch_types=[
            pltpu.VMEM((L * RPW,), jnp.int32),
            pltpu.VMEM((L * RPW,), jnp.float32),
            pltpu.VMEM((L * RPW,), jnp.float32),
            pltpu.VMEM((RPW,), jnp.float32),
            pltpu.VMEM((LANES,), jnp.float32),
            pltpu.SemaphoreType.DMA,
        ],
    )


@jax.jit
def kernel(inputs, mask, embed, W, b):
    s = _precompute_s(embed.astype(jnp.float32), W.astype(jnp.float32))
    idx_t, mask_t = _stage_t(
        inputs.astype(jnp.int32), mask.astype(jnp.float32))
    b16 = jnp.broadcast_to(b.astype(jnp.float32).reshape(()), (LANES,))
    return _make_sc_call()(s, idx_t, mask_t, b16)


# speedup vs baseline: 3.3850x; 1.0021x over previous
import functools

import jax
import jax.numpy as jnp
from jax import lax
from jax.experimental import pallas as pl
from jax.experimental.pallas import tpu as pltpu
from jax.experimental.pallas import tpu_sc as plsc

NC, NS, LANES = 2, 16, 16
NW = NC * NS

B, L = 16384, 50
D = 32
RPW = B // NW
CHUNKS = RPW // LANES
KROWS = (RPW * L) // 128


def _dot_body(e_ref, w_ref, o_ref):
    o_ref[:, :64] = jnp.sum(e_ref[...] * w_ref[...][None], axis=2)


SGRP = 64
SROWS_V = 1000000 // SGRP
SRB = 256
SGRID = (SROWS_V + SRB - 1) // SRB
SROWS = SGRID * SRB


def _precompute_s(embed, W):
    e3 = embed.reshape(SROWS_V, SGRP, D)
    s2 = pl.pallas_call(
        _dot_body,
        grid=(SGRID,),
        in_specs=[
            pl.BlockSpec((SRB, SGRP, D), lambda i: (i, 0, 0)),
            pl.BlockSpec((1, D), lambda i: (0, 0)),
        ],
        out_specs=pl.BlockSpec((SRB, 128), lambda i: (i, 0)),
        out_shape=jax.ShapeDtypeStruct((SROWS, 128), jnp.float32),
    )(e3, W)
    return s2.reshape(SROWS * 128)


SBLK = 1024
LP = 64


def _stage_body(i_ref, m_ref, oi_ref, om_ref):
    zi = jnp.zeros((128, LP - L), jnp.int32)
    zm = jnp.zeros((128, LP - L), jnp.float32)
    ics, mcs = [], []
    for g in range(SBLK // 128):
        sl = pl.ds(g * 128, 128)
        it = jnp.concatenate([i_ref[sl, :], zi], axis=1).T
        mt = jnp.concatenate([m_ref[sl, :], zm], axis=1).T
        it = ((it >> 6) << 7) | (it & 63)
        ics.append(it[:, None, :])
        mcs.append(mt[:, None, :])
    oi_ref[...] = jnp.concatenate(ics, axis=1)
    om_ref[...] = jnp.concatenate(mcs, axis=1)


def _stage_t(inputs, mask):
    grid = B // SBLK
    gb = SBLK // 128
    idxf, maskf = pl.pallas_call(
        _stage_body,
        grid=(grid,),
        in_specs=[
            pl.BlockSpec((SBLK, L), lambda i: (i, 0)),
            pl.BlockSpec((SBLK, L), lambda i: (i, 0)),
        ],
        out_specs=[
            pl.BlockSpec((LP, gb, 128), lambda i: (0, i, 0)),
            pl.BlockSpec((LP, gb, 128), lambda i: (0, i, 0)),
        ],
        out_shape=[
            jax.ShapeDtypeStruct((LP, B // 128, 128), jnp.int32),
            jax.ShapeDtypeStruct((LP, B // 128, 128), jnp.float32),
        ],
    )(inputs, mask)
    return idxf.reshape(LP * B), maskf.reshape(LP * B)


def _sc_body(s_hbm, idx_hbm, mask_hbm, b_hbm, out_hbm,
             idx_v, mask_v, vals_v, out_v, b_v, sem):
    wid = lax.axis_index("s") * NC + lax.axis_index("c")
    base = wid * RPW
    copies = []
    for l in range(L):
        copies.append(pltpu.async_copy(
            idx_hbm.at[pl.ds(l * B + base, RPW)],
            idx_v.at[pl.ds(l * RPW, RPW)], sem))
        copies.append(pltpu.async_copy(
            mask_hbm.at[pl.ds(l * B + base, RPW)],
            mask_v.at[pl.ds(l * RPW, RPW)], sem))
    pltpu.sync_copy(b_hbm, b_v)
    for cp in copies:
        cp.wait()
    pltpu.async_copy(s_hbm.at[idx_v], vals_v, sem).wait()

    bias = b_v[...]
    zero = jnp.zeros((LANES,), jnp.float32)
    for c in range(CHUNKS):
        col = c * LANES
        def body(l, carry, col=col):
            acc, msum = carry
            off = l * RPW + col
            v = vals_v[pl.ds(off, LANES)]
            m = mask_v[pl.ds(off, LANES)]
            return acc + v * m, msum + m
        acc, msum = lax.fori_loop(0, L, body, (zero, zero))
        out_v[pl.ds(c * LANES, LANES)] = acc / (msum + 1e-9) + bias
    pltpu.sync_copy(out_v, out_hbm.at[pl.ds(wid * RPW, RPW)])


@functools.cache
def _make_sc_call():
    mesh = plsc.VectorSubcoreMesh(
        core_axis_name="c", subcore_axis_name="s",
        num_cores=NC, num_subcores=NS)
    return pl.kernel(
        _sc_body,
        out_type=jax.ShapeDtypeStruct((B,), jnp.float32),
        mesh=mesh,
        compiler_params=pltpu.CompilerParams(needs_layout_passes=False),
        scratch_types=[
            pltpu.VMEM((L * RPW,), jnp.int32),
            pltpu.VMEM((L * RPW,), jnp.float32),
            pltpu.VMEM((L * RPW,), jnp.float32),
            pltpu.VMEM((RPW,), jnp.float32),
            pltpu.VMEM((LANES,), jnp.float32),
            pltpu.SemaphoreType.DMA,
        ],
    )


@jax.jit
def kernel(inputs, mask, embed, W, b):
    s = _precompute_s(embed.astype(jnp.float32), W.astype(jnp.float32))
    idx_t, mask_t = _stage_t(
        inputs.astype(jnp.int32), mask.astype(jnp.float32))
    b16 = jnp.broadcast_to(b.astype(jnp.float32).reshape(()), (LANES,))
    return _make_sc_call()(s, idx_t, mask_t, b16)


# speedup vs baseline: 9.9656x; 2.9441x over previous
import functools

import jax
import jax.numpy as jnp
from jax import lax
from jax.experimental import pallas as pl
from jax.experimental.pallas import tpu as pltpu
from jax.experimental.pallas import tpu_sc as plsc

NC, NS, LANES = 2, 16, 16
NW = NC * NS

B, L = 16384, 50
D = 32
RPW = B // NW
CHUNKS = RPW // LANES
KROWS = (RPW * L) // 128


def _dot_body(e_ref, w_ref, o_ref):
    o_ref[...] = jnp.sum(e_ref[...] * w_ref[...], axis=0)


SCB = 16384


def _precompute_s(embed_t, w_col):
    V = embed_t.shape[1]
    grid = (V + SCB - 1) // SCB
    return pl.pallas_call(
        _dot_body,
        grid=(grid,),
        in_specs=[
            pl.BlockSpec((D, SCB), lambda i: (0, i)),
            pl.BlockSpec((D, 1), lambda i: (0, 0)),
        ],
        out_specs=pl.BlockSpec((SCB,), lambda i: (i,)),
        out_shape=jax.ShapeDtypeStruct((V,), jnp.float32),
    )(embed_t, w_col)


SBLK = 1024
LP = 64


def _sc_body(s_hbm, idx_hbm, mask_hbm, b_hbm, out_hbm,
             idx_v, mask_v, vals_v, out_v, b_v, sem):
    wid = lax.axis_index("s") * NC + lax.axis_index("c")
    base = wid * RPW
    copies = []
    for l in range(L):
        copies.append(pltpu.async_copy(
            idx_hbm.at[pl.ds(l * B + base, RPW)],
            idx_v.at[pl.ds(l * RPW, RPW)], sem))
        copies.append(pltpu.async_copy(
            mask_hbm.at[pl.ds(l * B + base, RPW)],
            mask_v.at[pl.ds(l * RPW, RPW)], sem))
    pltpu.sync_copy(b_hbm, b_v)
    for cp in copies:
        cp.wait()
    pltpu.async_copy(s_hbm.at[idx_v], vals_v, sem).wait()

    bias = b_v[...]
    zero = jnp.zeros((LANES,), jnp.float32)
    for c in range(CHUNKS):
        col = c * LANES
        def body(l, carry, col=col):
            acc, msum = carry
            off = l * RPW + col
            v = vals_v[pl.ds(off, LANES)]
            m = mask_v[pl.ds(off, LANES)]
            return acc + v * m, msum + m
        acc, msum = lax.fori_loop(0, L, body, (zero, zero))
        out_v[pl.ds(c * LANES, LANES)] = acc / (msum + 1e-9) + bias
    pltpu.sync_copy(out_v, out_hbm.at[pl.ds(wid * RPW, RPW)])


@functools.cache
def _make_sc_call():
    mesh = plsc.VectorSubcoreMesh(
        core_axis_name="c", subcore_axis_name="s",
        num_cores=NC, num_subcores=NS)
    return pl.kernel(
        _sc_body,
        out_type=jax.ShapeDtypeStruct((B,), jnp.float32),
        mesh=mesh,
        compiler_params=pltpu.CompilerParams(needs_layout_passes=False),
        scratch_types=[
            pltpu.VMEM((L * RPW,), jnp.int32),
            pltpu.VMEM((L * RPW,), jnp.float32),
            pltpu.VMEM((L * RPW,), jnp.float32),
            pltpu.VMEM((RPW,), jnp.float32),
            pltpu.VMEM((LANES,), jnp.float32),
            pltpu.SemaphoreType.DMA,
        ],
    )


@jax.jit
def kernel(inputs, mask, embed, W, b):
    s = _precompute_s(embed.astype(jnp.float32).T,
                      W.astype(jnp.float32).reshape(D, 1))
    idx_t = inputs.astype(jnp.int32).T.reshape(L * B)
    mask_t = mask.astype(jnp.float32).T.reshape(L * B)
    b16 = jnp.broadcast_to(b.astype(jnp.float32).reshape(()), (LANES,))
    return _make_sc_call()(s, idx_t, mask_t, b16)


# speedup vs baseline: 11.4248x; 1.1464x over previous
import functools

import jax
import jax.numpy as jnp
from jax import lax
from jax.experimental import pallas as pl
from jax.experimental.pallas import tpu as pltpu
from jax.experimental.pallas import tpu_sc as plsc

NC, NS, LANES = 2, 16, 16
NW = NC * NS

B, L = 16384, 50
D = 32
RPW = B // NW
CHUNKS = RPW // LANES
KROWS = (RPW * L) // 128


def _dot_body(e_ref, w_ref, o_ref):
    o_ref[...] = jnp.sum(e_ref[...] * w_ref[...], axis=0)


SCB = 32768


def _precompute_s(embed_t, w_col):
    V = embed_t.shape[1]
    grid = (V + SCB - 1) // SCB
    return pl.pallas_call(
        _dot_body,
        grid=(grid,),
        in_specs=[
            pl.BlockSpec((D, SCB), lambda i: (0, i)),
            pl.BlockSpec((D, 1), lambda i: (0, 0)),
        ],
        out_specs=pl.BlockSpec((SCB,), lambda i: (i,)),
        out_shape=jax.ShapeDtypeStruct((V,), jnp.float32),
    )(embed_t, w_col)


SBLK = 1024
LP = 64


def _sc_body(s_hbm, idx_hbm, mask_hbm, b_hbm, out_hbm,
             idx_v, mask_v, vals_v, out_v, b_v, sem):
    wid = lax.axis_index("s") * NC + lax.axis_index("c")
    base = wid * RPW
    copies = []
    for l in range(L):
        copies.append(pltpu.async_copy(
            idx_hbm.at[pl.ds(l * B + base, RPW)],
            idx_v.at[pl.ds(l * RPW, RPW)], sem))
        copies.append(pltpu.async_copy(
            mask_hbm.at[pl.ds(l * B + base, RPW)],
            mask_v.at[pl.ds(l * RPW, RPW)], sem))
    pltpu.sync_copy(b_hbm, b_v)
    for cp in copies:
        cp.wait()
    HL = L // 2
    HN = HL * RPW
    g0 = pltpu.async_copy(
        s_hbm.at[idx_v.at[pl.ds(0, HN)]], vals_v.at[pl.ds(0, HN)], sem)
    g1 = pltpu.async_copy(
        s_hbm.at[idx_v.at[pl.ds(HN, HN)]], vals_v.at[pl.ds(HN, HN)], sem)
    g0.wait()

    bias = b_v[...]
    zero = jnp.zeros((LANES,), jnp.float32)

    def half(l_lo, l_hi, carries):
        res = []
        for c in range(CHUNKS):
            col = c * LANES
            def body(l, carry, col=col):
                acc, msum = carry
                off = l * RPW + col
                v = vals_v[pl.ds(off, LANES)]
                m = mask_v[pl.ds(off, LANES)]
                return acc + v * m, msum + m
            res.append(lax.fori_loop(l_lo, l_hi, body, carries[c]))
        return res

    part = half(0, HL, [(zero, zero)] * CHUNKS)
    g1.wait()
    part = half(HL, L, part)
    for c in range(CHUNKS):
        acc, msum = part[c]
        out_v[pl.ds(c * LANES, LANES)] = acc / (msum + 1e-9) + bias
    pltpu.sync_copy(out_v, out_hbm.at[pl.ds(wid * RPW, RPW)])


@functools.cache
def _make_sc_call():
    mesh = plsc.VectorSubcoreMesh(
        core_axis_name="c", subcore_axis_name="s",
        num_cores=NC, num_subcores=NS)
    return pl.kernel(
        _sc_body,
        out_type=jax.ShapeDtypeStruct((B,), jnp.float32),
        mesh=mesh,
        compiler_params=pltpu.CompilerParams(needs_layout_passes=False),
        scratch_types=[
            pltpu.VMEM((L * RPW,), jnp.int32),
            pltpu.VMEM((L * RPW,), jnp.float32),
            pltpu.VMEM((L * RPW,), jnp.float32),
            pltpu.VMEM((RPW,), jnp.float32),
            pltpu.VMEM((LANES,), jnp.float32),
            pltpu.SemaphoreType.DMA,
        ],
    )


@jax.jit
def kernel(inputs, mask, embed, W, b):
    s = _precompute_s(embed.astype(jnp.float32).T,
                      W.astype(jnp.float32).reshape(D, 1))
    idx_t = inputs.astype(jnp.int32).T.reshape(L * B)
    mask_t = mask.astype(jnp.float32).T.reshape(L * B)
    b16 = jnp.broadcast_to(b.astype(jnp.float32).reshape(()), (LANES,))
    return _make_sc_call()(s, idx_t, mask_t, b16)
